# Initial kernel scaffold; baseline (speedup 1.0000x reference)
import jax
import jax.numpy as jnp
from jax.experimental import pallas as pl


def kernel(x, edge_index, edge_value, params):
    raise NotImplementedError("write your pallas kernel here")


# speedup vs baseline: 4.6061x; 4.6061x over previous
import functools

import jax
import jax.numpy as jnp
from jax import lax
from jax.experimental import pallas as pl
from jax.experimental.pallas import tpu as pltpu
from jax.experimental.pallas import tpu_sc as plsc

N = 10000
NF = 64
E = 320000

NC = 2
NS = 16
NW = NC * NS
EPW = E // NW
K = 80
NCHUNK = EPW // K
K2 = 80
NCHUNK2 = EPW // K2
NPAD = 10240
RPW = NPAD // NS
ZROWS = 128

_MESH = plsc.VectorSubcoreMesh(
    core_axis_name="c", subcore_axis_name="s", num_cores=NC, num_subcores=NS)


def _sc_gather_body(table, idx, out, idx_v, rows_v, si0, si1, sg0, sg1,
                    so0, so1):
    wid = lax.axis_index("s") * NC + lax.axis_index("c")
    base = wid * EPW
    si = (si0, si1)
    sg = (sg0, sg1)
    so = (so0, so1)

    def idx_start(c, s):
        pltpu.async_copy(idx.at[pl.ds(base + c * K, K)], idx_v.at[s], si[s])

    def idx_wait(c, s):
        pltpu.make_async_copy(idx.at[pl.ds(base + c * K, K)], idx_v.at[s],
                              si[s]).wait()

    def g_start(c, s):
        del c
        pltpu.async_copy(table.at[idx_v.at[s]], rows_v.at[s], sg[s])

    def g_wait(c, s):
        del c
        pltpu.make_async_copy(table.at[idx_v.at[s]], rows_v.at[s],
                              sg[s]).wait()

    def out_start(c, s):
        pltpu.async_copy(rows_v.at[s], out.at[pl.ds(base + c * K, K)], so[s])

    def out_wait(c, s):
        pltpu.make_async_copy(rows_v.at[s], out.at[pl.ds(base + c * K, K)],
                              so[s]).wait()

    idx_start(0, 0)
    idx_start(1, 1)
    idx_wait(0, 0)
    g_start(0, 0)
    g_wait(0, 0)
    out_start(0, 0)
    idx_start(2, 0)
    idx_wait(1, 1)
    g_start(1, 1)

    @pl.loop(1, (NCHUNK - 1) // 2)
    def _steady(ii):
        for b in range(2):
            i = 2 * ii + b
            s = b
            ns = 1 - b
            g_wait(i - 1, ns)
            out_start(i - 1, ns)
            idx_start(i + 1, ns)
            idx_wait(i, s)
            out_wait(i - 2, s)
            g_start(i, s)

    i = NCHUNK - 1
    g_wait(i - 1, 1)
    out_start(i - 1, 1)
    idx_wait(i, 0)
    out_wait(i - 2, 0)
    g_start(i, 0)
    g_wait(i, 0)
    out_start(i, 0)
    out_wait(i - 1, 1)
    out_wait(i, 0)


def _sc_gather(table, idx):
    return pl.kernel(
        _sc_gather_body,
        out_type=jax.ShapeDtypeStruct((E, 128), jnp.float32),
        mesh=_MESH,
        scratch_types=[
            pltpu.VMEM((2, K), jnp.int32),
            pltpu.VMEM((2, K, 128), jnp.float32),
            pltpu.SemaphoreType.DMA,
            pltpu.SemaphoreType.DMA,
            pltpu.SemaphoreType.DMA,
            pltpu.SemaphoreType.DMA,
            pltpu.SemaphoreType.DMA,
            pltpu.SemaphoreType.DMA,
        ],
    )(table, idx)


def _zero_vmem(ref, rows, width):
    @pl.loop(0, rows)
    def _row(i):
        for j in range(width // 16):
            ref[i, pl.ds(j * 16, 16)] = jnp.zeros((16,), jnp.float32)


def _sc_scatter_body(data, idx, out, idx_v, data_v, zb, acc, si0, si1,
                     sd0, sd1, ss0, ss1):
    sid = lax.axis_index("s")
    cid = lax.axis_index("c")
    base = (sid * NC + cid) * EPW

    _zero_vmem(zb, ZROWS, NF)

    @pl.loop(0, RPW // ZROWS)
    def _z(r):
        pltpu.sync_copy(zb, acc.at[pl.ds(sid * RPW + r * ZROWS, ZROWS)])

    plsc.subcore_barrier()

    si = (si0, si1)
    sd = (sd0, sd1)
    ss = (ss0, ss1)

    def ld_start(c, s):
        pltpu.async_copy(idx.at[pl.ds(base + c * K2, K2)], idx_v.at[s], si[s])
        pltpu.async_copy(data.at[pl.ds(base + c * K2, K2)], data_v.at[s],
                         sd[s])

    def ld_wait(c, s):
        pltpu.make_async_copy(idx.at[pl.ds(base + c * K2, K2)], idx_v.at[s],
                              si[s]).wait()
        pltpu.make_async_copy(data.at[pl.ds(base + c * K2, K2)],
                              data_v.at[s], sd[s]).wait()

    def sc_start(s):
        pltpu.async_copy(data_v.at[s], acc.at[idx_v.at[s]], ss[s],
                         add=True)

    def sc_wait(s):
        pltpu.make_async_copy(data_v.at[s], acc.at[idx_v.at[s]],
                              ss[s]).wait()

    ld_start(0, 0)
    ld_wait(0, 0)
    ld_start(1, 1)
    sc_start(0)

    @pl.loop(0, (NCHUNK2 - 1) // 2)
    def _steady(ii):
        for b in range(2):
            i = 1 + 2 * ii + b
            s = 1 - b
            ns = 1 - s
            sc_wait(ns)
            @pl.when(i + 1 < NCHUNK2)
            def _():
                ld_start(i + 1, ns)
            ld_wait(i, s)
            sc_start(s)

    sc_wait((NCHUNK2 - 1) & 1)

    plsc.subcore_barrier()
    pltpu.sync_copy(acc.at[pl.ds(sid * RPW, RPW)],
                    out.at[pl.ds(cid * NPAD + sid * RPW, RPW)])


def _sc_scatter(data, idx):
    out = pl.kernel(
        _sc_scatter_body,
        out_type=jax.ShapeDtypeStruct((2 * NPAD, NF), jnp.float32),
        mesh=_MESH,
        scratch_types=[
            pltpu.VMEM((2, K2), jnp.int32),
            pltpu.VMEM((2, K2, NF), jnp.float32),
            pltpu.VMEM((ZROWS, NF), jnp.float32),
            pltpu.VMEM_SHARED((NPAD, NF), jnp.float32),
            pltpu.SemaphoreType.DMA,
            pltpu.SemaphoreType.DMA,
            pltpu.SemaphoreType.DMA,
            pltpu.SemaphoreType.DMA,
            pltpu.SemaphoreType.DMA,
            pltpu.SemaphoreType.DMA,
        ],
        compiler_params=pltpu.CompilerParams(use_tc_tiling_on_sc=False),
    )(data, idx)
    return out.reshape(2, NPAD, NF)[:, :N]


def _sc_deg_body(idx, outdeg, idx_v, ones_v, zb16, accdeg):
    sid = lax.axis_index("s")
    cid = lax.axis_index("c")
    base = (sid * NC + cid) * EPW

    _zero_vmem(zb16, ZROWS, 16)

    @pl.loop(0, K)
    def _ones(i):
        ones_v[i, :] = jnp.ones((16,), jnp.float32)

    @pl.loop(0, RPW // ZROWS)
    def _z(r):
        pltpu.sync_copy(zb16, accdeg.at[pl.ds(sid * RPW + r * ZROWS, ZROWS)])

    plsc.subcore_barrier()

    @pl.loop(0, NCHUNK)
    def _chunk(c):
        off = base + c * K
        pltpu.sync_copy(idx.at[pl.ds(off, K)], idx_v.at[0])
        pltpu.sync_copy(ones_v, accdeg.at[idx_v.at[0]], add=True)

    plsc.subcore_barrier()
    pltpu.sync_copy(accdeg.at[pl.ds(sid * RPW, RPW)],
                    outdeg.at[pl.ds(cid * NPAD + sid * RPW, RPW)])


def _sc_deg(idx):
    outdeg = pl.kernel(
        _sc_deg_body,
        out_type=jax.ShapeDtypeStruct((2 * NPAD, 16), jnp.float32),
        mesh=_MESH,
        scratch_types=[
            pltpu.VMEM((1, K), jnp.int32),
            pltpu.VMEM((K, 16), jnp.float32),
            pltpu.VMEM((ZROWS, 16), jnp.float32),
            pltpu.VMEM_SHARED((NPAD, 16), jnp.float32),
        ],
        compiler_params=pltpu.CompilerParams(use_tc_tiling_on_sc=False),
    )(idx)
    return outdeg.reshape(2, NPAD, 16)[:, :N]


EB = 4000
EBG = E // EB
NB = 2000
NBG = N // NB

_f32 = jnp.float32


def _onehot(dst_ref):
    d = dst_ref[...]
    return (d == lax.broadcasted_iota(jnp.int32, (EB, NF), 1)).astype(_f32)


def _dot(a, b):
    return jnp.dot(a, b, preferred_element_type=_f32)


def _dott(a, b):
    return lax.dot_general(a, b, (((0,), (0,)), ((), ())),
                           preferred_element_type=_f32)


def _l0_body(ev, dst, a0, wmf, wmo, p0row, m_of, aggf, degd):
    b = pl.program_id(0)
    oh = _onehot(dst)
    evb = ev[...]
    m_of[...] = jnp.maximum(_dot(oh, a0[...]) + evb * wmf[...], 0.0)
    mfo = jnp.maximum(p0row[...] + evb * wmo[...], 0.0)

    @pl.when(b == 0)
    def _():
        aggf[...] = jnp.zeros_like(aggf)
        degd[...] = jnp.zeros_like(degd)

    aggf[...] += _dott(oh, mfo)
    degd[...] += _dott(oh, jnp.ones((EB, 1), _f32))


def _run_l0(ev, dst, a0, wmf, wmo, p0row):
    small = pl.BlockSpec((1, NF), lambda b: (0, 0))
    return pl.pallas_call(
        _l0_body,
        grid=(EBG,),
        in_specs=[
            pl.BlockSpec((EB, 1), lambda b: (b, 0)),
            pl.BlockSpec((EB, 1), lambda b: (b, 0)),
            pl.BlockSpec((NF, NF), lambda b: (0, 0)),
            small, small, small,
        ],
        out_specs=[
            pl.BlockSpec((EB, NF), lambda b: (b, 0)),
            pl.BlockSpec((NF, NF), lambda b: (0, 0)),
            pl.BlockSpec((NF, 1), lambda b: (0, 0)),
        ],
        out_shape=[
            jax.ShapeDtypeStruct((E, NF), _f32),
            jax.ShapeDtypeStruct((NF, NF), _f32),
            jax.ShapeDtypeStruct((NF, 1), _f32),
        ],
    )(ev, dst, a0, wmf, wmo, p0row)


def _edge0_body(ev, dst, gg, we0, b0, a1, wmf1, wmo1,
                e1_out, m_of, aggf):
    b = pl.program_id(0)
    oh = _onehot(dst)
    evb = ev[...]
    g = gg[...]
    ne = jnp.maximum(evb * we0[...] + g[:, :NF] + _dot(oh, b0[...]), 0.0)
    e1 = jnp.maximum(ne + evb, 0.0)
    e1_out[...] = e1
    m_of[...] = jnp.maximum(_dot(oh, a1[...]) + _dot(e1, wmf1[...]), 0.0)
    mfo = jnp.maximum(g[:, NF:] + _dot(e1, wmo1[...]), 0.0)

    @pl.when(b == 0)
    def _():
        aggf[...] = jnp.zeros_like(aggf)

    aggf[...] += _dott(oh, mfo)


def _run_edge0(ev, dst, gg, we0, b0, a1, wmf1, wmo1):
    sq = pl.BlockSpec((NF, NF), lambda b: (0, 0))
    return pl.pallas_call(
        _edge0_body,
        grid=(EBG,),
        in_specs=[
            pl.BlockSpec((EB, 1), lambda b: (b, 0)),
            pl.BlockSpec((EB, 1), lambda b: (b, 0)),
            pl.BlockSpec((EB, 2 * NF), lambda b: (b, 0)),
            pl.BlockSpec((1, NF), lambda b: (0, 0)),
            sq, sq, sq, sq,
        ],
        out_specs=[
            pl.BlockSpec((EB, NF), lambda b: (b, 0)),
            pl.BlockSpec((EB, NF), lambda b: (b, 0)),
            pl.BlockSpec((NF, NF), lambda b: (0, 0)),
        ],
        out_shape=[
            jax.ShapeDtypeStruct((E, NF), _f32),
            jax.ShapeDtypeStruct((E, NF), _f32),
            jax.ShapeDtypeStruct((NF, NF), _f32),
        ],
    )(ev, dst, gg, we0, b0, a1, wmf1, wmo1)


def _edge1_body(e1, dst, gg, we1, b1, a2, wmf2, wmo2, m_of, aggf):
    b = pl.program_id(0)
    oh = _onehot(dst)
    e1v = e1[...]
    g = gg[...]
    ne = jnp.maximum(_dot(e1v, we1[...]) + g[:, :NF] + _dot(oh, b1[...]),
                     0.0)
    e2 = jnp.maximum(ne + e1v, 0.0)
    m_of[...] = jnp.maximum(_dot(oh, a2[...]) + _dot(e2, wmf2[...]), 0.0)
    mfo = jnp.maximum(g[:, NF:] + _dot(e2, wmo2[...]), 0.0)

    @pl.when(b == 0)
    def _():
        aggf[...] = jnp.zeros_like(aggf)

    aggf[...] += _dott(oh, mfo)


def _run_edge1(e1, dst, gg, we1, b1, a2, wmf2, wmo2):
    sq = pl.BlockSpec((NF, NF), lambda b: (0, 0))
    return pl.pallas_call(
        _edge1_body,
        grid=(EBG,),
        in_specs=[
            pl.BlockSpec((EB, NF), lambda b: (b, 0)),
            pl.BlockSpec((EB, 1), lambda b: (b, 0)),
            pl.BlockSpec((EB, 2 * NF), lambda b: (b, 0)),
            sq, sq, sq, sq, sq,
        ],
        out_specs=[
            pl.BlockSpec((EB, NF), lambda b: (b, 0)),
            pl.BlockSpec((NF, NF), lambda b: (0, 0)),
        ],
        out_shape=[
            jax.ShapeDtypeStruct((E, NF), _f32),
            jax.ShapeDtypeStruct((NF, NF), _f32),
        ],
    )(e1, dst, gg, we1, b1, a2, wmf2, wmo2)


def _eye():
    r = lax.broadcasted_iota(jnp.int32, (NF, NF), 0)
    c = lax.broadcasted_iota(jnp.int32, (NF, NF), 1)
    return (r == c).astype(_f32)


def _make_dense_body(layer0, last):
    def body(*refs):
        (aggo, deg, degd, aggf, vin, fin, wn1, wn2, bn, wev, wmov_n, bmo_n,
         wf1, wf2, bf, wef, be, wmff_n, bmf_n, *outs) = refs
        b = pl.program_id(0)
        a = aggo[...]
        d = deg[...]
        cnt = jnp.maximum(d[0, :, 0:1] + d[1, :, 0:1], 1.0)
        ao = (a[0] + a[1]) / cnt
        if layer0:
            nn = jnp.maximum(wn1[...] + _dot(ao, wn2[...]) + bn[...], 0.0)
            vn = jnp.maximum(nn + 1.0, 0.0)
        else:
            v = vin[...]
            nn = jnp.maximum(_dot(v, wn1[...]) + _dot(ao, wn2[...]) + bn[...],
                             0.0)
            vn = jnp.maximum(nn + v, 0.0)
        if last:
            h_out, fht_out = outs
            h_out[...] = _dot(vn, wev[...])
        else:
            vn_out, g_out, fn_out, b_out, an_out = outs
            vn_out[...] = vn
            g_out[...] = jnp.concatenate(
                [_dot(nn, wev[...]), _dot(vn, wmov_n[...]) + bmo_n[...]],
                axis=1)

        @pl.when(b == 0)
        def _():
            cntd = jnp.maximum(degd[...], 1.0)
            af = aggf[...] / cntd
            if layer0:
                nf = jnp.maximum(wf1[...] + _dot(af, wf2[...]) + bf[...], 0.0)
                fn = jnp.maximum(nf + _eye(), 0.0)
            else:
                f = fin[...]
                nf = jnp.maximum(_dot(f, wf1[...]) + _dot(af, wf2[...])
                                 + bf[...], 0.0)
                fn = jnp.maximum(nf + f, 0.0)
            if last:
                fht_out[...] = lax.dot_general(
                    wef[...], fn, (((0,), (1,)), ((), ())),
                    preferred_element_type=_f32) + be[...]
            else:
                b_out[...] = _dot(nf, wef[...]) + be[...]
                an_out[...] = _dot(fn, wmff_n[...]) + bmf_n[...]
                fn_out[...] = fn

    return body


def _run_dense(layer0, last, aggo, deg, degd, aggf, vin, fin, wn1, wn2, bn,
               wev, wmov_n, bmo_n, wf1, wf2, bf, wef, be, wmff_n, bmf_n):
    sq = pl.BlockSpec((NF, NF), lambda b: (0, 0))
    row = pl.BlockSpec((1, NF), lambda b: (0, 0))
    col = pl.BlockSpec((NF, 1), lambda b: (0, 0))
    nblk = pl.BlockSpec((NB, NF), lambda b: (b, 0))
    in_specs = [
        pl.BlockSpec((2, NB, NF), lambda b: (0, b, 0)),
        pl.BlockSpec((2, NB, 16), lambda b: (0, b, 0)),
        col, sq,
        sq if layer0 else nblk,
        sq,
        row if layer0 else sq,
        sq, row, sq, sq, row, sq, sq, row, sq,
        col if last else row,
        sq, row,
    ]
    if last:
        out_specs = [nblk, sq]
        out_shape = [
            jax.ShapeDtypeStruct((N, NF), _f32),
            jax.ShapeDtypeStruct((NF, NF), _f32),
        ]
    else:
        out_specs = [
            nblk,
            pl.BlockSpec((NB, 2 * NF), lambda b: (b, 0)),
            sq, sq, sq,
        ]
        out_shape = [
            jax.ShapeDtypeStruct((N, NF), _f32),
            jax.ShapeDtypeStruct((N, 2 * NF), _f32),
            jax.ShapeDtypeStruct((NF, NF), _f32),
            jax.ShapeDtypeStruct((NF, NF), _f32),
            jax.ShapeDtypeStruct((NF, NF), _f32),
        ]
    return pl.pallas_call(
        _make_dense_body(layer0, last),
        grid=(NBG,),
        in_specs=in_specs,
        out_specs=out_specs,
        out_shape=out_shape,
    )(aggo, deg, degd, aggf, vin, fin, wn1, wn2, bn, wev, wmov_n, bmo_n,
      wf1, wf2, bf, wef, be, wmff_n, bmf_n)


def _head_body(h, fht, wout, bout, w1, b1, w2, b2, dhat, yhat):
    hv = h[...]
    fh = fht[...]
    wv = wout[...]
    acc = jnp.zeros((NB, NF), _f32)
    for k in range(NF):
        t = jnp.maximum(hv[:, k:k + 1] + fh[k:k + 1, :], 0.0)
        acc = acc + t * wv[0:1, k:k + 1]
    d = acc + bout[...]
    dhat[...] = d
    y = jnp.maximum(_dot(d, w1[...]) + b1[...], 0.0)
    yhat[...] = _dot(y, w2[...]) + b2[...]


def _run_head(h, fht, wout, bout, w1, b1, w2, b2):
    sq = pl.BlockSpec((NF, NF), lambda b: (0, 0))
    row = pl.BlockSpec((1, NF), lambda b: (0, 0))
    one = pl.BlockSpec((1, 1), lambda b: (0, 0))
    return pl.pallas_call(
        _head_body,
        grid=(NBG,),
        in_specs=[
            pl.BlockSpec((NB, NF), lambda b: (b, 0)),
            sq, row, one, sq, row,
            pl.BlockSpec((NF, 1), lambda b: (0, 0)),
            one,
        ],
        out_specs=[
            pl.BlockSpec((NB, NF), lambda b: (b, 0)),
            pl.BlockSpec((NB, 1), lambda b: (b, 0)),
        ],
        out_shape=[
            jax.ShapeDtypeStruct((N, NF), _f32),
            jax.ShapeDtypeStruct((N, 1), _f32),
        ],
    )(h, fht, wout, bout, w1, b1, w2, b2)


@jax.jit
def kernel(x, edge_index, edge_value, params):
    del x
    src = edge_index[0]
    dst2 = edge_index[1][:, None]
    ev2 = edge_value[:, None]

    p0, p1, p2 = params['block0'], params['block1'], params['block2']
    ep, npar = params['eph'], params['nph']

    a0 = p0['Wmf'][:NF] + p0['bmf'][None, :]
    wmf0_e = p0['Wmf'][NF:NF + 1]
    wmo0_e = p0['Wmo'][NF:NF + 1]
    p0row = (p0['Wmo'][:NF].sum(axis=0) + p0['bmo'])[None, :]
    wn1_0 = p0['Wn'][:NF].sum(axis=0)[None, :]
    we0_e = p0['We'][0:1]
    we0_v = p0['We'][1:1 + NF]
    we0_f = p0['We'][1 + NF:]

    m_of0, aggf0, degd = _run_l0(ev2, dst2, a0, wmf0_e, wmo0_e, p0row)

    deg = _sc_deg(src)
    aggo0 = _sc_scatter(m_of0, src)

    dummy = jnp.zeros((NF, NF), _f32)
    v1, g0, f1, b0t, a1 = _run_dense(
        True, False, aggo0, deg, degd, aggf0, dummy, dummy,
        wn1_0, p0['Wn'][NF:], p0['bn'][None, :],
        we0_v, p1['Wmo'][:NF], p1['bmo'][None, :],
        p0['Wf'][:NF], p0['Wf'][NF:], p0['bf'][None, :],
        we0_f, p0['be'][None, :],
        p1['Wmf'][:NF], p1['bmf'][None, :])

    g0g = _sc_gather(g0, src)

    e1, m_of1, aggf1 = _run_edge0(
        ev2, dst2, g0g, we0_e, b0t, a1,
        p1['Wmf'][NF:], p1['Wmo'][NF:])

    aggo1 = _sc_scatter(m_of1, src)

    v2, g1, f2, b1t, a2 = _run_dense(
        False, False, aggo1, deg, degd, aggf1, v1, f1,
        p1['Wn'][:NF], p1['Wn'][NF:], p1['bn'][None, :],
        p1['We'][NF:2 * NF], p2['Wmo'][:NF], p2['bmo'][None, :],
        p1['Wf'][:NF], p1['Wf'][NF:], p1['bf'][None, :],
        p1['We'][2 * NF:], p1['be'][None, :],
        p2['Wmf'][:NF], p2['bmf'][None, :])

    g1g = _sc_gather(g1, src)

    m_of2, aggf2 = _run_edge1(
        e1, dst2, g1g, p1['We'][:NF], b1t, a2, p2['Wmf'][NF:],
        p2['Wmo'][NF:])

    aggo2 = _sc_scatter(m_of2, src)

    h, fht = _run_dense(
        False, True, aggo2, deg, degd, aggf2, v2, f2,
        p2['Wn'][:NF], p2['Wn'][NF:], p2['bn'][None, :],
        ep['Wo'], dummy, jnp.zeros((1, NF), _f32),
        p2['Wf'][:NF], p2['Wf'][NF:], p2['bf'][None, :],
        ep['Wf'], ep['bh'][:, None],
        dummy, jnp.zeros((1, NF), _f32))

    d_hat, y_hat = _run_head(
        h, fht, ep['wout'][None, :], ep['bout'][None, None],
        npar['W1'], npar['b1'][None, :], npar['W2'], npar['b2'][None, :])

    return d_hat, y_hat
